# baseline (device time: 234327 ns/iter reference)
import jax
import jax.numpy as jnp
from jax import lax
from jax.experimental import pallas as pl
from jax.experimental.pallas import tpu as pltpu

BLK = 256
LAG = 8
SEND_SLOTS = 8

_CompilerParams = getattr(pltpu, "CompilerParams", None) or getattr(
    pltpu, "TPUCompilerParams"
)
_DeviceIdType = getattr(pl, "DeviceIdType", None) or getattr(pltpu, "DeviceIdType")
_sem_signal = getattr(pl, "semaphore_signal", None) or getattr(
    pltpu, "semaphore_signal"
)
_sem_wait = getattr(pl, "semaphore_wait", None) or getattr(pltpu, "semaphore_wait")


def kernel(O, Wo):
    B, S, H, D = O.shape
    K = H * D
    N = Wo.shape[1]
    R = S // 2
    NB = N // BLK
    RING = LAG + 1

    O2 = O.reshape(S, K).astype(jnp.bfloat16)

    def body(o_ref, wo_ref, out_ref, send_buf, recv_buf, mine_ring,
             send_sems, recv_sems):
        t = pl.program_id(0)
        x = lax.axis_index("x")
        y = lax.axis_index("y")
        z = lax.axis_index("z")
        peer = (1 - x, y, z)
        is0 = x == 0

        def rdma(slot, chunk):
            return pltpu.make_async_remote_copy(
                src_ref=send_buf.at[slot],
                dst_ref=recv_buf.at[chunk],
                send_sem=send_sems.at[slot],
                recv_sem=recv_sems.at[chunk],
                device_id=peer,
                device_id_type=_DeviceIdType.MESH,
            )

        @pl.when(t == 0)
        def _():
            barrier = pltpu.get_barrier_semaphore()
            _sem_signal(barrier, inc=1, device_id=peer,
                        device_id_type=_DeviceIdType.MESH)
            _sem_wait(barrier, 1)

        @pl.when(t < NB)
        def _():
            wt = wo_ref[...].astype(jnp.bfloat16)
            p_top = jnp.dot(
                o_ref[:R, :], wt, preferred_element_type=jnp.float32
            ).astype(jnp.bfloat16)
            p_bot = jnp.dot(
                o_ref[R:, :], wt, preferred_element_type=jnp.float32
            ).astype(jnp.bfloat16)

            slot = t % SEND_SLOTS

            @pl.when(t >= SEND_SLOTS)
            def _():
                rdma(slot, 0).wait_send()

            ring = t % RING

            @pl.when(is0)
            def _():
                mine_ring[ring] = p_top
                send_buf[slot] = p_bot

            @pl.when(jnp.logical_not(is0))
            def _():
                mine_ring[ring] = p_bot
                send_buf[slot] = p_top

            rdma(slot, t).start()

        @pl.when(t >= LAG)
        def _():
            c = t - LAG
            rdma(0, c).wait_recv()
            out_ref[0] = (
                mine_ring[c % RING].astype(jnp.float32)
                + recv_buf[c].astype(jnp.float32)
            ).astype(jnp.bfloat16)

        @pl.when(t == NB + LAG - 1)
        def _():
            for i in range(SEND_SLOTS):
                rdma(i, 0).wait_send()

    out = pl.pallas_call(
        body,
        grid=(NB + LAG,),
        out_shape=jax.ShapeDtypeStruct((B, R, N), jnp.bfloat16),
        in_specs=[
            pl.BlockSpec(memory_space=pltpu.MemorySpace.VMEM),
            pl.BlockSpec((K, BLK), lambda t: (0, jnp.minimum(t, NB - 1))),
        ],
        out_specs=pl.BlockSpec(
            (B, R, BLK), lambda t: (0, 0, jnp.clip(t - LAG, 0, NB - 1))
        ),
        scratch_shapes=[
            pltpu.VMEM((SEND_SLOTS, R, BLK), jnp.bfloat16),
            pltpu.VMEM((NB, R, BLK), jnp.bfloat16),
            pltpu.VMEM((RING, R, BLK), jnp.bfloat16),
            pltpu.SemaphoreType.DMA((SEND_SLOTS,)),
            pltpu.SemaphoreType.DMA((NB,)),
        ],
        compiler_params=_CompilerParams(
            dimension_semantics=("arbitrary",),
            collective_id=0,
            vmem_limit_bytes=int(62.9 * 1024 * 1024),
        ),
    )(O2, Wo)

    return out


# device time: 180265 ns/iter; 1.2999x vs baseline; 1.2999x over previous
import jax
import jax.numpy as jnp
from jax import lax
from jax.experimental import pallas as pl
from jax.experimental.pallas import tpu as pltpu

BLK = 256
LAGX = 6
SEND_SLOTS = 8

_CompilerParams = getattr(pltpu, "CompilerParams", None) or getattr(
    pltpu, "TPUCompilerParams"
)
_DeviceIdType = getattr(pl, "DeviceIdType", None) or getattr(pltpu, "DeviceIdType")
_sem_signal = getattr(pl, "semaphore_signal", None) or getattr(
    pltpu, "semaphore_signal"
)
_sem_wait = getattr(pl, "semaphore_wait", None) or getattr(pltpu, "semaphore_wait")


def kernel(O, Wo):
    B, S, H, D = O.shape
    K = H * D
    N = Wo.shape[1]
    R = S // 2
    NBH = (N // BLK) // 2

    O2 = O.reshape(S, K).astype(jnp.bfloat16)
    base = jnp.reshape(lax.axis_index("y") * NBH, (1,)).astype(jnp.int32)

    def body(s_ref, o_ref, wo_ref, out_ref, send_buf, recv_x,
             xsend_sems, xrecv_sems, ysend_sems, yrecv_sems):
        t = pl.program_id(0)
        x = lax.axis_index("x")
        y = lax.axis_index("y")
        z = lax.axis_index("z")
        xpeer = (1 - x, y, z)
        ypeer = (x, 1 - y, z)
        is0 = x == 0
        col0 = s_ref[0] * BLK

        def rdma_x(slot, chunk):
            return pltpu.make_async_remote_copy(
                src_ref=send_buf.at[slot],
                dst_ref=recv_x.at[chunk],
                send_sem=xsend_sems.at[slot],
                recv_sem=xrecv_sems.at[chunk],
                device_id=xpeer,
                device_id_type=_DeviceIdType.MESH,
            )

        def rdma_y(chunk, col):
            return pltpu.make_async_remote_copy(
                src_ref=out_ref.at[0, :, pl.ds(col, BLK)],
                dst_ref=out_ref.at[0, :, pl.ds(col, BLK)],
                send_sem=ysend_sems.at[chunk],
                recv_sem=yrecv_sems.at[chunk],
                device_id=ypeer,
                device_id_type=_DeviceIdType.MESH,
            )

        @pl.when(t == 0)
        def _():
            barrier = pltpu.get_barrier_semaphore()
            for nbr in (xpeer, ypeer):
                _sem_signal(barrier, inc=1, device_id=nbr,
                            device_id_type=_DeviceIdType.MESH)
            _sem_wait(barrier, 2)

        @pl.when(t < NBH)
        def _():
            wt = wo_ref[...].astype(jnp.bfloat16)
            p_top = jnp.dot(
                o_ref[:R, :], wt, preferred_element_type=jnp.float32
            ).astype(jnp.bfloat16)
            p_bot = jnp.dot(
                o_ref[R:, :], wt, preferred_element_type=jnp.float32
            ).astype(jnp.bfloat16)
            col = col0 + t * BLK
            slot = t % SEND_SLOTS

            @pl.when(t >= SEND_SLOTS)
            def _():
                rdma_x(slot, 0).wait_send()

            @pl.when(is0)
            def _():
                out_ref[0, :, pl.ds(col, BLK)] = p_top
                send_buf[slot] = p_bot

            @pl.when(jnp.logical_not(is0))
            def _():
                out_ref[0, :, pl.ds(col, BLK)] = p_bot
                send_buf[slot] = p_top

            rdma_x(slot, t).start()

        @pl.when(t >= LAGX)
        def _():
            c = t - LAGX
            col = col0 + c * BLK
            rdma_x(0, c).wait_recv()
            out_ref[0, :, pl.ds(col, BLK)] = (
                out_ref[0, :, pl.ds(col, BLK)].astype(jnp.float32)
                + recv_x[c].astype(jnp.float32)
            ).astype(jnp.bfloat16)
            rdma_y(c, col).start()

        @pl.when(t == NBH + LAGX - 1)
        def _():
            for i in range(SEND_SLOTS):
                rdma_x(i, 0).wait_send()
            for i in range(NBH):
                rdma_y(i, 0).wait_send()
            for i in range(NBH):
                rdma_y(i, 0).wait_recv()

    out = pl.pallas_call(
        body,
        grid_spec=pltpu.PrefetchScalarGridSpec(
            num_scalar_prefetch=1,
            grid=(NBH + LAGX,),
            in_specs=[
                pl.BlockSpec(memory_space=pltpu.MemorySpace.VMEM),
                pl.BlockSpec(
                    (K, BLK), lambda t, s: (0, s[0] + jnp.minimum(t, NBH - 1))
                ),
            ],
            out_specs=pl.BlockSpec(memory_space=pltpu.MemorySpace.VMEM),
            scratch_shapes=[
                pltpu.VMEM((SEND_SLOTS, R, BLK), jnp.bfloat16),
                pltpu.VMEM((NBH, R, BLK), jnp.bfloat16),
                pltpu.SemaphoreType.DMA((SEND_SLOTS,)),
                pltpu.SemaphoreType.DMA((NBH,)),
                pltpu.SemaphoreType.DMA((NBH,)),
                pltpu.SemaphoreType.DMA((NBH,)),
            ],
        ),
        out_shape=jax.ShapeDtypeStruct((B, R, N), jnp.bfloat16),
        compiler_params=_CompilerParams(
            dimension_semantics=("arbitrary",),
            collective_id=0,
            vmem_limit_bytes=int(62.9 * 1024 * 1024),
        ),
    )(base, O2, Wo)

    return out


# device time: 175563 ns/iter; 1.3347x vs baseline; 1.0268x over previous
import jax
import jax.numpy as jnp
from jax import lax
from jax.experimental import pallas as pl
from jax.experimental.pallas import tpu as pltpu

BLK = 256
LAGX = 6
SEND_SLOTS = 8

_CompilerParams = getattr(pltpu, "CompilerParams", None) or getattr(
    pltpu, "TPUCompilerParams"
)
_DeviceIdType = getattr(pl, "DeviceIdType", None) or getattr(pltpu, "DeviceIdType")
_sem_signal = getattr(pl, "semaphore_signal", None) or getattr(
    pltpu, "semaphore_signal"
)
_sem_wait = getattr(pl, "semaphore_wait", None) or getattr(pltpu, "semaphore_wait")


def kernel(O, Wo):
    B, S, H, D = O.shape
    K = H * D
    N = Wo.shape[1]
    R = S // 2
    NBH = (N // BLK) // 2

    O2 = O.reshape(S, K).astype(jnp.bfloat16)
    base = jnp.reshape(lax.axis_index("y") * NBH, (1,)).astype(jnp.int32)

    def body(s_ref, o_ref, wo_ref, out_ref, send_buf, recv_x,
             xsend_sems, xrecv_sems, ysend_sems, yrecv_sems):
        t = pl.program_id(0)
        x = lax.axis_index("x")
        y = lax.axis_index("y")
        z = lax.axis_index("z")
        xpeer = (1 - x, y, z)
        ypeer = (x, 1 - y, z)
        is0 = x == 0
        col0 = s_ref[0] * BLK

        def rdma_x(slot, chunk):
            return pltpu.make_async_remote_copy(
                src_ref=send_buf.at[slot],
                dst_ref=recv_x.at[chunk],
                send_sem=xsend_sems.at[slot],
                recv_sem=xrecv_sems.at[chunk],
                device_id=xpeer,
                device_id_type=_DeviceIdType.MESH,
            )

        def rdma_y(chunk, col):
            return pltpu.make_async_remote_copy(
                src_ref=out_ref.at[0, :, pl.ds(col, BLK)],
                dst_ref=out_ref.at[0, :, pl.ds(col, BLK)],
                send_sem=ysend_sems.at[chunk],
                recv_sem=yrecv_sems.at[chunk],
                device_id=ypeer,
                device_id_type=_DeviceIdType.MESH,
            )

        @pl.when(t == 0)
        def _():
            barrier = pltpu.get_barrier_semaphore()
            for nbr in (xpeer, ypeer):
                _sem_signal(barrier, inc=1, device_id=nbr,
                            device_id_type=_DeviceIdType.MESH)
            _sem_wait(barrier, 2)

        @pl.when(t >= LAGX)
        def _():
            c = t - LAGX
            col = col0 + c * BLK
            rdma_x(0, c).wait_recv()
            out_ref[0, :, pl.ds(col, BLK)] = (
                out_ref[0, :, pl.ds(col, BLK)] + recv_x[c]
            )
            rdma_y(c, col).start()

        @pl.when(t < NBH)
        def _():
            wt = wo_ref[...].astype(jnp.bfloat16)
            p_top = jnp.dot(
                o_ref[:R, :], wt, preferred_element_type=jnp.float32
            ).astype(jnp.bfloat16)
            p_bot = jnp.dot(
                o_ref[R:, :], wt, preferred_element_type=jnp.float32
            ).astype(jnp.bfloat16)
            col = col0 + t * BLK
            slot = t % SEND_SLOTS

            @pl.when(t >= SEND_SLOTS)
            def _():
                rdma_x(slot, 0).wait_send()

            @pl.when(is0)
            def _():
                out_ref[0, :, pl.ds(col, BLK)] = p_top
                send_buf[slot] = p_bot

            @pl.when(jnp.logical_not(is0))
            def _():
                out_ref[0, :, pl.ds(col, BLK)] = p_bot
                send_buf[slot] = p_top

            rdma_x(slot, t).start()

        @pl.when(t == NBH + LAGX - 1)
        def _():
            for i in range(SEND_SLOTS):
                rdma_x(i, 0).wait_send()
            for i in range(NBH):
                rdma_y(i, 0).wait_send()
            for i in range(NBH):
                rdma_y(i, 0).wait_recv()

    out = pl.pallas_call(
        body,
        grid_spec=pltpu.PrefetchScalarGridSpec(
            num_scalar_prefetch=1,
            grid=(NBH + LAGX,),
            in_specs=[
                pl.BlockSpec(memory_space=pltpu.MemorySpace.VMEM),
                pl.BlockSpec(
                    (K, BLK), lambda t, s: (0, s[0] + jnp.minimum(t, NBH - 1))
                ),
            ],
            out_specs=pl.BlockSpec(memory_space=pltpu.MemorySpace.VMEM),
            scratch_shapes=[
                pltpu.VMEM((SEND_SLOTS, R, BLK), jnp.bfloat16),
                pltpu.VMEM((NBH, R, BLK), jnp.bfloat16),
                pltpu.SemaphoreType.DMA((SEND_SLOTS,)),
                pltpu.SemaphoreType.DMA((NBH,)),
                pltpu.SemaphoreType.DMA((NBH,)),
                pltpu.SemaphoreType.DMA((NBH,)),
            ],
        ),
        out_shape=jax.ShapeDtypeStruct((B, R, N), jnp.bfloat16),
        compiler_params=_CompilerParams(
            dimension_semantics=("arbitrary",),
            collective_id=0,
            vmem_limit_bytes=int(62.9 * 1024 * 1024),
        ),
    )(base, O2, Wo)

    return out


# device time: 175402 ns/iter; 1.3359x vs baseline; 1.0009x over previous
import jax
import jax.numpy as jnp
from jax import lax
from jax.experimental import pallas as pl
from jax.experimental.pallas import tpu as pltpu

BLK = 512
LAGX = 3
SEND_SLOTS = 2

_CompilerParams = getattr(pltpu, "CompilerParams", None) or getattr(
    pltpu, "TPUCompilerParams"
)
_DeviceIdType = getattr(pl, "DeviceIdType", None) or getattr(pltpu, "DeviceIdType")
_sem_signal = getattr(pl, "semaphore_signal", None) or getattr(
    pltpu, "semaphore_signal"
)
_sem_wait = getattr(pl, "semaphore_wait", None) or getattr(pltpu, "semaphore_wait")


def kernel(O, Wo):
    B, S, H, D = O.shape
    K = H * D
    N = Wo.shape[1]
    R = S // 2
    NBH = (N // BLK) // 2

    O2 = O.reshape(S, K).astype(jnp.bfloat16)
    base = jnp.reshape(lax.axis_index("y") * NBH, (1,)).astype(jnp.int32)

    def body(s_ref, o_ref, wo_ref, out_ref, send_buf, recv_x,
             xsend_sems, xrecv_sems, ysend_sems, yrecv_sems):
        t = pl.program_id(0)
        x = lax.axis_index("x")
        y = lax.axis_index("y")
        z = lax.axis_index("z")
        xpeer = (1 - x, y, z)
        ypeer = (x, 1 - y, z)
        is0 = x == 0
        col0 = s_ref[0] * BLK

        def rdma_x(slot, chunk):
            return pltpu.make_async_remote_copy(
                src_ref=send_buf.at[slot],
                dst_ref=recv_x.at[chunk],
                send_sem=xsend_sems.at[slot],
                recv_sem=xrecv_sems.at[chunk],
                device_id=xpeer,
                device_id_type=_DeviceIdType.MESH,
            )

        def rdma_y(chunk, col):
            return pltpu.make_async_remote_copy(
                src_ref=out_ref.at[0, :, pl.ds(col, BLK)],
                dst_ref=out_ref.at[0, :, pl.ds(col, BLK)],
                send_sem=ysend_sems.at[chunk],
                recv_sem=yrecv_sems.at[chunk],
                device_id=ypeer,
                device_id_type=_DeviceIdType.MESH,
            )

        @pl.when(t == 0)
        def _():
            barrier = pltpu.get_barrier_semaphore()
            for nbr in (xpeer, ypeer):
                _sem_signal(barrier, inc=1, device_id=nbr,
                            device_id_type=_DeviceIdType.MESH)
            _sem_wait(barrier, 2)

        @pl.when(t >= LAGX)
        def _():
            c = t - LAGX
            col = col0 + c * BLK
            rdma_x(0, c).wait_recv()
            out_ref[0, :, pl.ds(col, BLK)] = (
                out_ref[0, :, pl.ds(col, BLK)] + recv_x[c]
            )
            rdma_y(c, col).start()

        @pl.when(t < NBH)
        def _():
            wt = wo_ref[...].astype(jnp.bfloat16)
            p_top = jnp.dot(
                o_ref[:R, :], wt, preferred_element_type=jnp.float32
            ).astype(jnp.bfloat16)
            p_bot = jnp.dot(
                o_ref[R:, :], wt, preferred_element_type=jnp.float32
            ).astype(jnp.bfloat16)
            col = col0 + t * BLK
            slot = t % SEND_SLOTS

            @pl.when(t >= SEND_SLOTS)
            def _():
                rdma_x(slot, 0).wait_send()

            @pl.when(is0)
            def _():
                out_ref[0, :, pl.ds(col, BLK)] = p_top
                send_buf[slot] = p_bot

            @pl.when(jnp.logical_not(is0))
            def _():
                out_ref[0, :, pl.ds(col, BLK)] = p_bot
                send_buf[slot] = p_top

            rdma_x(slot, t).start()

        @pl.when(t == NBH + LAGX - 1)
        def _():
            for i in range(SEND_SLOTS):
                rdma_x(i, 0).wait_send()
            for i in range(NBH):
                rdma_y(i, 0).wait_send()
            for i in range(NBH):
                rdma_y(i, 0).wait_recv()

    out = pl.pallas_call(
        body,
        grid_spec=pltpu.PrefetchScalarGridSpec(
            num_scalar_prefetch=1,
            grid=(NBH + LAGX,),
            in_specs=[
                pl.BlockSpec(memory_space=pltpu.MemorySpace.VMEM),
                pl.BlockSpec(
                    (K, BLK), lambda t, s: (0, s[0] + jnp.minimum(t, NBH - 1))
                ),
            ],
            out_specs=pl.BlockSpec(memory_space=pltpu.MemorySpace.VMEM),
            scratch_shapes=[
                pltpu.VMEM((SEND_SLOTS, R, BLK), jnp.bfloat16),
                pltpu.VMEM((NBH, R, BLK), jnp.bfloat16),
                pltpu.SemaphoreType.DMA((SEND_SLOTS,)),
                pltpu.SemaphoreType.DMA((NBH,)),
                pltpu.SemaphoreType.DMA((NBH,)),
                pltpu.SemaphoreType.DMA((NBH,)),
            ],
        ),
        out_shape=jax.ShapeDtypeStruct((B, R, N), jnp.bfloat16),
        compiler_params=_CompilerParams(
            dimension_semantics=("arbitrary",),
            collective_id=0,
            vmem_limit_bytes=int(63.8 * 1024 * 1024),
        ),
    )(base, O2, Wo)

    return out


# device time: 166867 ns/iter; 1.4043x vs baseline; 1.0511x over previous
import jax
import jax.numpy as jnp
from jax import lax
from jax.experimental import pallas as pl
from jax.experimental.pallas import tpu as pltpu

BLK = 256
LAGX = 4
SEND_SLOTS = 4

_CompilerParams = getattr(pltpu, "CompilerParams", None) or getattr(
    pltpu, "TPUCompilerParams"
)
_DeviceIdType = getattr(pl, "DeviceIdType", None) or getattr(pltpu, "DeviceIdType")
_sem_signal = getattr(pl, "semaphore_signal", None) or getattr(
    pltpu, "semaphore_signal"
)
_sem_wait = getattr(pl, "semaphore_wait", None) or getattr(pltpu, "semaphore_wait")


def kernel(O, Wo):
    B, S, H, D = O.shape
    K = H * D
    N = Wo.shape[1]
    R = S // 2
    NBH = (N // BLK) // 4

    O2 = O.reshape(S, K).astype(jnp.bfloat16)
    quarter = lax.axis_index("y") * 2 + (lax.axis_index("z") & 1)
    base = jnp.reshape(quarter * NBH, (1,)).astype(jnp.int32)

    def body(s_ref, o_ref, wo_ref, out_ref, send_buf, recv_x,
             xsend_sems, xrecv_sems, ysend_sems, yrecv_sems,
             zsend_sems, zrecv_sems, dsend_sems, drecv_sems):
        t = pl.program_id(0)
        x = lax.axis_index("x")
        y = lax.axis_index("y")
        z = lax.axis_index("z")
        zp = z ^ 1
        xpeer = (1 - x, y, z)
        gather_peers = (
            ((x, 1 - y, z), ysend_sems, yrecv_sems),
            ((x, y, zp), zsend_sems, zrecv_sems),
            ((x, 1 - y, zp), dsend_sems, drecv_sems),
        )
        is0 = x == 0
        col0 = s_ref[0] * BLK

        def rdma_x(slot, chunk):
            return pltpu.make_async_remote_copy(
                src_ref=send_buf.at[slot],
                dst_ref=recv_x.at[chunk],
                send_sem=xsend_sems.at[slot],
                recv_sem=xrecv_sems.at[chunk],
                device_id=xpeer,
                device_id_type=_DeviceIdType.MESH,
            )

        def rdma_g(peer_idx, chunk, col):
            dev, ssems, rsems = gather_peers[peer_idx]
            return pltpu.make_async_remote_copy(
                src_ref=out_ref.at[0, :, pl.ds(col, BLK)],
                dst_ref=out_ref.at[0, :, pl.ds(col, BLK)],
                send_sem=ssems.at[chunk],
                recv_sem=rsems.at[chunk],
                device_id=dev,
                device_id_type=_DeviceIdType.MESH,
            )

        @pl.when(t == 0)
        def _():
            barrier = pltpu.get_barrier_semaphore()
            for nbr in (xpeer,) + tuple(p[0] for p in gather_peers):
                _sem_signal(barrier, inc=1, device_id=nbr,
                            device_id_type=_DeviceIdType.MESH)
            _sem_wait(barrier, 4)

        @pl.when(t >= LAGX)
        def _():
            c = t - LAGX
            col = col0 + c * BLK
            rdma_x(0, c).wait_recv()
            out_ref[0, :, pl.ds(col, BLK)] = (
                out_ref[0, :, pl.ds(col, BLK)] + recv_x[c]
            )
            for g in range(3):
                rdma_g(g, c, col).start()

        @pl.when(t < NBH)
        def _():
            wt = wo_ref[...].astype(jnp.bfloat16)
            p_top = jnp.dot(
                o_ref[:R, :], wt, preferred_element_type=jnp.float32
            ).astype(jnp.bfloat16)
            p_bot = jnp.dot(
                o_ref[R:, :], wt, preferred_element_type=jnp.float32
            ).astype(jnp.bfloat16)
            col = col0 + t * BLK
            slot = t % SEND_SLOTS

            @pl.when(t >= SEND_SLOTS)
            def _():
                rdma_x(slot, 0).wait_send()

            @pl.when(is0)
            def _():
                out_ref[0, :, pl.ds(col, BLK)] = p_top
                send_buf[slot] = p_bot

            @pl.when(jnp.logical_not(is0))
            def _():
                out_ref[0, :, pl.ds(col, BLK)] = p_bot
                send_buf[slot] = p_top

            rdma_x(slot, t).start()

        @pl.when(t == NBH + LAGX - 1)
        def _():
            for i in range(SEND_SLOTS):
                rdma_x(i, 0).wait_send()
            for g in range(3):
                for i in range(NBH):
                    rdma_g(g, i, 0).wait_send()
            for g in range(3):
                for i in range(NBH):
                    rdma_g(g, i, 0).wait_recv()

    out = pl.pallas_call(
        body,
        grid_spec=pltpu.PrefetchScalarGridSpec(
            num_scalar_prefetch=1,
            grid=(NBH + LAGX,),
            in_specs=[
                pl.BlockSpec(memory_space=pltpu.MemorySpace.VMEM),
                pl.BlockSpec(
                    (K, BLK), lambda t, s: (0, s[0] + jnp.minimum(t, NBH - 1))
                ),
            ],
            out_specs=pl.BlockSpec(memory_space=pltpu.MemorySpace.VMEM),
            scratch_shapes=[
                pltpu.VMEM((SEND_SLOTS, R, BLK), jnp.bfloat16),
                pltpu.VMEM((NBH, R, BLK), jnp.bfloat16),
                pltpu.SemaphoreType.DMA((SEND_SLOTS,)),
                pltpu.SemaphoreType.DMA((NBH,)),
                pltpu.SemaphoreType.DMA((NBH,)),
                pltpu.SemaphoreType.DMA((NBH,)),
                pltpu.SemaphoreType.DMA((NBH,)),
                pltpu.SemaphoreType.DMA((NBH,)),
                pltpu.SemaphoreType.DMA((NBH,)),
                pltpu.SemaphoreType.DMA((NBH,)),
            ],
        ),
        out_shape=jax.ShapeDtypeStruct((B, R, N), jnp.bfloat16),
        compiler_params=_CompilerParams(
            dimension_semantics=("arbitrary",),
            collective_id=0,
            vmem_limit_bytes=int(62.9 * 1024 * 1024),
        ),
    )(base, O2, Wo)

    return out


# device time: 161926 ns/iter; 1.4471x vs baseline; 1.0305x over previous
import jax
import jax.numpy as jnp
from jax import lax
from jax.experimental import pallas as pl
from jax.experimental.pallas import tpu as pltpu

BLK = 256
LAGX = 3
SEND_SLOTS = 4

_CompilerParams = getattr(pltpu, "CompilerParams", None) or getattr(
    pltpu, "TPUCompilerParams"
)
_DeviceIdType = getattr(pl, "DeviceIdType", None) or getattr(pltpu, "DeviceIdType")
_sem_signal = getattr(pl, "semaphore_signal", None) or getattr(
    pltpu, "semaphore_signal"
)
_sem_wait = getattr(pl, "semaphore_wait", None) or getattr(pltpu, "semaphore_wait")


def kernel(O, Wo):
    B, S, H, D = O.shape
    K = H * D
    N = Wo.shape[1]
    R = S // 2
    NBH = (N // BLK) // 4

    O2 = O.reshape(S, K).astype(jnp.bfloat16)
    quarter = lax.axis_index("y") * 2 + (lax.axis_index("z") & 1)
    base = jnp.reshape(quarter * NBH, (1,)).astype(jnp.int32)

    def body(s_ref, o_ref, wo_ref, out_ref, send_buf, recv_x,
             xsend_sems, xrecv_sems, ysend_sems, yrecv_sems,
             zsend_sems, zrecv_sems, dsend_sems, drecv_sems):
        t = pl.program_id(0)
        x = lax.axis_index("x")
        y = lax.axis_index("y")
        z = lax.axis_index("z")
        zp = z ^ 1
        xpeer = (1 - x, y, z)
        gather_peers = (
            ((x, 1 - y, z), ysend_sems, yrecv_sems),
            ((x, y, zp), zsend_sems, zrecv_sems),
            ((x, 1 - y, zp), dsend_sems, drecv_sems),
        )
        is0 = x == 0
        col0 = s_ref[0] * BLK

        def rdma_x(slot, chunk):
            return pltpu.make_async_remote_copy(
                src_ref=send_buf.at[slot],
                dst_ref=recv_x.at[chunk],
                send_sem=xsend_sems.at[slot],
                recv_sem=xrecv_sems.at[chunk],
                device_id=xpeer,
                device_id_type=_DeviceIdType.MESH,
            )

        def rdma_g(peer_idx, chunk, col):
            dev, ssems, rsems = gather_peers[peer_idx]
            return pltpu.make_async_remote_copy(
                src_ref=out_ref.at[0, :, pl.ds(col, BLK)],
                dst_ref=out_ref.at[0, :, pl.ds(col, BLK)],
                send_sem=ssems.at[chunk],
                recv_sem=rsems.at[chunk],
                device_id=dev,
                device_id_type=_DeviceIdType.MESH,
            )

        @pl.when(t == 0)
        def _():
            barrier = pltpu.get_barrier_semaphore()
            for nbr in (xpeer,) + tuple(p[0] for p in gather_peers):
                _sem_signal(barrier, inc=1, device_id=nbr,
                            device_id_type=_DeviceIdType.MESH)
            _sem_wait(barrier, 4)

        @pl.when(t >= LAGX)
        def _():
            c = t - LAGX
            col = col0 + c * BLK
            rdma_x(0, c).wait_recv()
            out_ref[0, :, pl.ds(col, BLK)] = (
                out_ref[0, :, pl.ds(col, BLK)] + recv_x[c]
            )
            for g in range(3):
                rdma_g(g, c, col).start()

        @pl.when(t < NBH)
        def _():
            wt = wo_ref[...].astype(jnp.bfloat16)
            p_top = jnp.dot(
                o_ref[:R, :], wt, preferred_element_type=jnp.float32
            ).astype(jnp.bfloat16)
            p_bot = jnp.dot(
                o_ref[R:, :], wt, preferred_element_type=jnp.float32
            ).astype(jnp.bfloat16)
            col = col0 + t * BLK
            slot = t % SEND_SLOTS

            @pl.when(t >= SEND_SLOTS)
            def _():
                rdma_x(slot, 0).wait_send()

            @pl.when(is0)
            def _():
                out_ref[0, :, pl.ds(col, BLK)] = p_top
                send_buf[slot] = p_bot

            @pl.when(jnp.logical_not(is0))
            def _():
                out_ref[0, :, pl.ds(col, BLK)] = p_bot
                send_buf[slot] = p_top

            rdma_x(slot, t).start()

        @pl.when(t == NBH + LAGX - 1)
        def _():
            for i in range(SEND_SLOTS):
                rdma_x(i, 0).wait_send()
            for g in range(3):
                for i in range(NBH):
                    rdma_g(g, i, 0).wait_send()
            for g in range(3):
                for i in range(NBH):
                    rdma_g(g, i, 0).wait_recv()

    out = pl.pallas_call(
        body,
        grid_spec=pltpu.PrefetchScalarGridSpec(
            num_scalar_prefetch=1,
            grid=(NBH + LAGX,),
            in_specs=[
                pl.BlockSpec(memory_space=pltpu.MemorySpace.VMEM),
                pl.BlockSpec(
                    (K, BLK), lambda t, s: (0, s[0] + jnp.minimum(t, NBH - 1))
                ),
            ],
            out_specs=pl.BlockSpec(memory_space=pltpu.MemorySpace.VMEM),
            scratch_shapes=[
                pltpu.VMEM((SEND_SLOTS, R, BLK), jnp.bfloat16),
                pltpu.VMEM((NBH, R, BLK), jnp.bfloat16),
                pltpu.SemaphoreType.DMA((SEND_SLOTS,)),
                pltpu.SemaphoreType.DMA((NBH,)),
                pltpu.SemaphoreType.DMA((NBH,)),
                pltpu.SemaphoreType.DMA((NBH,)),
                pltpu.SemaphoreType.DMA((NBH,)),
                pltpu.SemaphoreType.DMA((NBH,)),
                pltpu.SemaphoreType.DMA((NBH,)),
                pltpu.SemaphoreType.DMA((NBH,)),
            ],
        ),
        out_shape=jax.ShapeDtypeStruct((B, R, N), jnp.bfloat16),
        compiler_params=_CompilerParams(
            dimension_semantics=("arbitrary",),
            collective_id=0,
            vmem_limit_bytes=int(62.9 * 1024 * 1024),
        ),
    )(base, O2, Wo)

    return out


# device time: 158724 ns/iter; 1.4763x vs baseline; 1.0202x over previous
import jax
import jax.numpy as jnp
from jax import lax
from jax.experimental import pallas as pl
from jax.experimental.pallas import tpu as pltpu

BLK = 256
LAGX = 2
SEND_SLOTS = 4

_CompilerParams = getattr(pltpu, "CompilerParams", None) or getattr(
    pltpu, "TPUCompilerParams"
)
_DeviceIdType = getattr(pl, "DeviceIdType", None) or getattr(pltpu, "DeviceIdType")
_sem_signal = getattr(pl, "semaphore_signal", None) or getattr(
    pltpu, "semaphore_signal"
)
_sem_wait = getattr(pl, "semaphore_wait", None) or getattr(pltpu, "semaphore_wait")


def kernel(O, Wo):
    B, S, H, D = O.shape
    K = H * D
    N = Wo.shape[1]
    R = S // 2
    NBH = (N // BLK) // 4

    O2 = O.reshape(S, K).astype(jnp.bfloat16)
    quarter = lax.axis_index("y") * 2 + (lax.axis_index("z") & 1)
    base = jnp.reshape(quarter * NBH, (1,)).astype(jnp.int32)

    def body(s_ref, o_ref, wo_ref, out_ref, send_buf, recv_x,
             xsend_sems, xrecv_sems, ysend_sems, yrecv_sems,
             zsend_sems, zrecv_sems, dsend_sems, drecv_sems):
        t = pl.program_id(0)
        x = lax.axis_index("x")
        y = lax.axis_index("y")
        z = lax.axis_index("z")
        zp = z ^ 1
        xpeer = (1 - x, y, z)
        gather_peers = (
            ((x, 1 - y, z), ysend_sems, yrecv_sems),
            ((x, y, zp), zsend_sems, zrecv_sems),
            ((x, 1 - y, zp), dsend_sems, drecv_sems),
        )
        is0 = x == 0
        col0 = s_ref[0] * BLK

        def rdma_x(slot, chunk):
            return pltpu.make_async_remote_copy(
                src_ref=send_buf.at[slot],
                dst_ref=recv_x.at[chunk],
                send_sem=xsend_sems.at[slot],
                recv_sem=xrecv_sems.at[chunk],
                device_id=xpeer,
                device_id_type=_DeviceIdType.MESH,
            )

        def rdma_g(peer_idx, chunk, col):
            dev, ssems, rsems = gather_peers[peer_idx]
            return pltpu.make_async_remote_copy(
                src_ref=out_ref.at[0, :, pl.ds(col, BLK)],
                dst_ref=out_ref.at[0, :, pl.ds(col, BLK)],
                send_sem=ssems.at[chunk],
                recv_sem=rsems.at[chunk],
                device_id=dev,
                device_id_type=_DeviceIdType.MESH,
            )

        @pl.when(t == 0)
        def _():
            barrier = pltpu.get_barrier_semaphore()
            for nbr in (xpeer,) + tuple(p[0] for p in gather_peers):
                _sem_signal(barrier, inc=1, device_id=nbr,
                            device_id_type=_DeviceIdType.MESH)
            _sem_wait(barrier, 4)

        @pl.when(t >= LAGX)
        def _():
            c = t - LAGX
            col = col0 + c * BLK
            rdma_x(0, c).wait_recv()
            out_ref[0, :, pl.ds(col, BLK)] = (
                out_ref[0, :, pl.ds(col, BLK)] + recv_x[c]
            )
            for g in range(3):
                rdma_g(g, c, col).start()

        @pl.when(t < NBH)
        def _():
            wt = wo_ref[...].astype(jnp.bfloat16)
            p_top = jnp.dot(
                o_ref[:R, :], wt, preferred_element_type=jnp.float32
            ).astype(jnp.bfloat16)
            p_bot = jnp.dot(
                o_ref[R:, :], wt, preferred_element_type=jnp.float32
            ).astype(jnp.bfloat16)
            col = col0 + t * BLK
            slot = t % SEND_SLOTS

            @pl.when(t >= SEND_SLOTS)
            def _():
                rdma_x(slot, 0).wait_send()

            @pl.when(is0)
            def _():
                out_ref[0, :, pl.ds(col, BLK)] = p_top
                send_buf[slot] = p_bot

            @pl.when(jnp.logical_not(is0))
            def _():
                out_ref[0, :, pl.ds(col, BLK)] = p_bot
                send_buf[slot] = p_top

            rdma_x(slot, t).start()

        @pl.when(t == NBH + LAGX - 1)
        def _():
            for i in range(SEND_SLOTS):
                rdma_x(i, 0).wait_send()
            for g in range(3):
                for i in range(NBH):
                    rdma_g(g, i, 0).wait_send()
            for g in range(3):
                for i in range(NBH):
                    rdma_g(g, i, 0).wait_recv()

    out = pl.pallas_call(
        body,
        grid_spec=pltpu.PrefetchScalarGridSpec(
            num_scalar_prefetch=1,
            grid=(NBH + LAGX,),
            in_specs=[
                pl.BlockSpec(memory_space=pltpu.MemorySpace.VMEM),
                pl.BlockSpec(
                    (K, BLK), lambda t, s: (0, s[0] + jnp.minimum(t, NBH - 1))
                ),
            ],
            out_specs=pl.BlockSpec(memory_space=pltpu.MemorySpace.VMEM),
            scratch_shapes=[
                pltpu.VMEM((SEND_SLOTS, R, BLK), jnp.bfloat16),
                pltpu.VMEM((NBH, R, BLK), jnp.bfloat16),
                pltpu.SemaphoreType.DMA((SEND_SLOTS,)),
                pltpu.SemaphoreType.DMA((NBH,)),
                pltpu.SemaphoreType.DMA((NBH,)),
                pltpu.SemaphoreType.DMA((NBH,)),
                pltpu.SemaphoreType.DMA((NBH,)),
                pltpu.SemaphoreType.DMA((NBH,)),
                pltpu.SemaphoreType.DMA((NBH,)),
                pltpu.SemaphoreType.DMA((NBH,)),
            ],
        ),
        out_shape=jax.ShapeDtypeStruct((B, R, N), jnp.bfloat16),
        compiler_params=_CompilerParams(
            dimension_semantics=("arbitrary",),
            collective_id=0,
            vmem_limit_bytes=int(62.9 * 1024 * 1024),
        ),
    )(base, O2, Wo)

    return out
